# baseline (speedup 1.0000x reference)
import functools

import jax
import jax.numpy as jnp
from jax import lax
from jax.experimental import pallas as pl
from jax.experimental.pallas import tpu as pltpu
from jax.experimental.pallas import tpu_sc as plsc

D_MODEL = 2048
LANES = 16
NUM_WORKERS = 32
K = 16


@functools.partial(jax.jit, static_argnames=())
def _gather_add(seq_tok, struct_tok, seq_table, struct_table):
    n = seq_tok.shape[0]
    per_w = n // NUM_WORKERS
    n_chunks = per_w // K
    mesh = plsc.VectorSubcoreMesh(core_axis_name="c", subcore_axis_name="s")

    @functools.partial(
        pl.kernel,
        mesh=mesh,
        out_type=jax.ShapeDtypeStruct((n, D_MODEL), jnp.float32),
        scratch_types=[
            pltpu.VMEM((per_w,), jnp.int32),
            pltpu.VMEM((per_w,), jnp.int32),
            pltpu.VMEM((K, D_MODEL), jnp.float32),
            pltpu.VMEM((K, D_MODEL), jnp.float32),
            pltpu.SemaphoreType.DMA,
            pltpu.SemaphoreType.DMA,
        ],
    )
    def k(seq_tok_hbm, struct_tok_hbm, seq_tab_hbm, struct_tab_hbm, out_hbm,
          sidx, tidx, buf_a, buf_b, sem_a, sem_b):
        wid = lax.axis_index("s") * 2 + lax.axis_index("c")
        base = wid * per_w
        pltpu.sync_copy(seq_tok_hbm.at[pl.ds(base, per_w)], sidx)
        pltpu.sync_copy(struct_tok_hbm.at[pl.ds(base, per_w)], tidx)

        def chunk_body(c, _):
            off = c * K
            cp_a = pltpu.async_copy(
                seq_tab_hbm.at[sidx.at[pl.ds(off, K)]], buf_a, sem_a)
            cp_b = pltpu.async_copy(
                struct_tab_hbm.at[tidx.at[pl.ds(off, K)]], buf_b, sem_b)
            cp_a.wait()
            cp_b.wait()

            def row_body(i, _):
                def vec_body(j, _):
                    sl = pl.ds(j * LANES, LANES)
                    buf_a[i, sl] = buf_a[i, sl] + buf_b[i, sl]
                    return 0
                lax.fori_loop(0, D_MODEL // LANES, vec_body, 0)
                return 0
            lax.fori_loop(0, K, row_body, 0)

            pltpu.sync_copy(buf_a, out_hbm.at[pl.ds(base + off, K)])
            return 0

        lax.fori_loop(0, n_chunks, chunk_body, 0)

    return k(seq_tok, struct_tok, seq_table, struct_table)


def kernel(sequence_tokens, structure_tokens, seq_table, struct_table):
    b, s = sequence_tokens.shape
    n = b * s
    seq_tok = sequence_tokens.reshape(n).astype(jnp.int32)
    struct_tok = structure_tokens.reshape(n).astype(jnp.int32)
    out = _gather_add(seq_tok, struct_tok, seq_table, struct_table)
    return out.reshape(b, s, D_MODEL)


# speedup vs baseline: 1.8550x; 1.8550x over previous
import functools

import jax
import jax.numpy as jnp
from jax import lax
from jax.experimental import pallas as pl
from jax.experimental.pallas import tpu as pltpu
from jax.experimental.pallas import tpu_sc as plsc

D_MODEL = 2048
LANES = 16
NUM_WORKERS = 32
K = 8
NBUF = 2


@jax.jit
def _gather_add(seq_tok, struct_tok, seq_table, struct_table):
    n = seq_tok.shape[0]
    per_w = n // NUM_WORKERS
    n_chunks = per_w // K
    n_outer = n_chunks // NBUF
    mesh = plsc.VectorSubcoreMesh(core_axis_name="c", subcore_axis_name="s")

    @functools.partial(
        pl.kernel,
        mesh=mesh,
        out_type=jax.ShapeDtypeStruct((n, D_MODEL), jnp.float32),
        scratch_types=[
            pltpu.VMEM((per_w,), jnp.int32),
            pltpu.VMEM((per_w,), jnp.int32),
            pltpu.VMEM((NBUF, K, D_MODEL), jnp.float32),
            pltpu.VMEM((NBUF, K, D_MODEL), jnp.float32),
            pltpu.VMEM((NBUF, K, D_MODEL), jnp.float32),
            pltpu.SemaphoreType.DMA,
            pltpu.SemaphoreType.DMA,
            pltpu.SemaphoreType.DMA,
            pltpu.SemaphoreType.DMA,
            pltpu.SemaphoreType.DMA,
            pltpu.SemaphoreType.DMA,
        ],
    )
    def k(seq_tok_hbm, struct_tok_hbm, seq_tab_hbm, struct_tab_hbm, out_hbm,
          sidx, tidx, buf_s, buf_t, buf_o,
          sem_s0, sem_s1, sem_t0, sem_t1, sem_w0, sem_w1):
        sem_s = (sem_s0, sem_s1)
        sem_t = (sem_t0, sem_t1)
        sem_w = (sem_w0, sem_w1)
        wid = lax.axis_index("s") * 2 + lax.axis_index("c")
        base = wid * per_w
        pltpu.sync_copy(seq_tok_hbm.at[pl.ds(base, per_w)], sidx)
        pltpu.sync_copy(struct_tok_hbm.at[pl.ds(base, per_w)], tidx)

        for b in range(NBUF):
            off = b * K
            pltpu.async_copy(
                seq_tab_hbm.at[sidx.at[pl.ds(off, K)]], buf_s.at[b], sem_s[b])
            pltpu.async_copy(
                struct_tab_hbm.at[tidx.at[pl.ds(off, K)]], buf_t.at[b], sem_t[b])

        def outer(o, _):
            for b in range(NBUF):
                off = (o * NBUF + b) * K
                pltpu.make_async_copy(
                    seq_tab_hbm.at[sidx.at[pl.ds(off, K)]], buf_s.at[b],
                    sem_s[b]).wait()
                pltpu.make_async_copy(
                    struct_tab_hbm.at[tidx.at[pl.ds(off, K)]], buf_t.at[b],
                    sem_t[b]).wait()
                @pl.when(o > 0)
                def _():
                    pltpu.make_async_copy(
                        buf_o.at[b],
                        out_hbm.at[pl.ds(base + off - NBUF * K, K)],
                        sem_w[b]).wait()
                for i in range(K):
                    def vec_body(j, _):
                        sl = pl.ds(j * LANES, LANES)
                        buf_o[b, i, sl] = buf_s[b, i, sl] + buf_t[b, i, sl]
                        return 0
                    lax.fori_loop(0, D_MODEL // LANES, vec_body, 0)
                pltpu.async_copy(
                    buf_o.at[b], out_hbm.at[pl.ds(base + off, K)], sem_w[b])
                @pl.when(o < n_outer - 1)
                def _():
                    off2 = off + NBUF * K
                    pltpu.async_copy(
                        seq_tab_hbm.at[sidx.at[pl.ds(off2, K)]], buf_s.at[b],
                        sem_s[b])
                    pltpu.async_copy(
                        struct_tab_hbm.at[tidx.at[pl.ds(off2, K)]], buf_t.at[b],
                        sem_t[b])
            return 0

        lax.fori_loop(0, n_outer, outer, 0)

        for b in range(NBUF):
            off = (n_chunks - NBUF + b) * K
            pltpu.make_async_copy(
                buf_o.at[b], out_hbm.at[pl.ds(base + off, K)], sem_w[b]).wait()

    return k(seq_tok, struct_tok, seq_table, struct_table)


def kernel(sequence_tokens, structure_tokens, seq_table, struct_table):
    b, s = sequence_tokens.shape
    n = b * s
    seq_tok = sequence_tokens.reshape(n).astype(jnp.int32)
    struct_tok = structure_tokens.reshape(n).astype(jnp.int32)
    out = _gather_add(seq_tok, struct_tok, seq_table, struct_table)
    return out.reshape(b, s, D_MODEL)


# speedup vs baseline: 2.2650x; 1.2210x over previous
import functools

import jax
import jax.numpy as jnp
from jax import lax
from jax.experimental import pallas as pl
from jax.experimental.pallas import tpu as pltpu
from jax.experimental.pallas import tpu_sc as plsc

D_MODEL = 2048
LANES = 16
NUM_WORKERS = 32
K = 8
NBUF = 2


@jax.jit
def _gather_add(seq_tok, struct_tok, seq_table, struct_table):
    n = seq_tok.shape[0]
    per_w = n // NUM_WORKERS
    n_chunks = per_w // K
    n_outer = n_chunks // NBUF
    mesh = plsc.VectorSubcoreMesh(core_axis_name="c", subcore_axis_name="s")

    @functools.partial(
        pl.kernel,
        mesh=mesh,
        out_type=jax.ShapeDtypeStruct((n, D_MODEL), jnp.float32),
        scratch_types=[
            pltpu.VMEM((per_w,), jnp.int32),
            pltpu.VMEM((per_w,), jnp.int32),
            pltpu.VMEM((NBUF, K, D_MODEL), jnp.float32),
            pltpu.VMEM((NBUF, K, D_MODEL), jnp.float32),
            pltpu.VMEM((NBUF, K, D_MODEL), jnp.float32),
            pltpu.SemaphoreType.DMA,
            pltpu.SemaphoreType.DMA,
            pltpu.SemaphoreType.DMA,
            pltpu.SemaphoreType.DMA,
            pltpu.SemaphoreType.DMA,
            pltpu.SemaphoreType.DMA,
        ],
    )
    def k(seq_tok_hbm, struct_tok_hbm, seq_tab_hbm, struct_tab_hbm, out_hbm,
          sidx, tidx, buf_s, buf_t, buf_o,
          sem_s0, sem_s1, sem_t0, sem_t1, sem_w0, sem_w1):
        sem_s = (sem_s0, sem_s1)
        sem_t = (sem_t0, sem_t1)
        sem_w = (sem_w0, sem_w1)
        wid = lax.axis_index("s") * 2 + lax.axis_index("c")
        base = wid * per_w
        pltpu.sync_copy(seq_tok_hbm.at[pl.ds(base, per_w)], sidx)
        pltpu.sync_copy(struct_tok_hbm.at[pl.ds(base, per_w)], tidx)

        for b in range(NBUF):
            off = b * K
            pltpu.async_copy(
                seq_tab_hbm.at[sidx.at[pl.ds(off, K)]], buf_s.at[b], sem_s[b])
            pltpu.async_copy(
                struct_tab_hbm.at[tidx.at[pl.ds(off, K)]], buf_t.at[b], sem_t[b])

        def outer(o, _):
            for b in range(NBUF):
                off = (o * NBUF + b) * K
                pltpu.make_async_copy(
                    seq_tab_hbm.at[sidx.at[pl.ds(off, K)]], buf_s.at[b],
                    sem_s[b]).wait()
                pltpu.make_async_copy(
                    struct_tab_hbm.at[tidx.at[pl.ds(off, K)]], buf_t.at[b],
                    sem_t[b]).wait()
                @pl.when(o > 0)
                def _():
                    pltpu.make_async_copy(
                        buf_o.at[b],
                        out_hbm.at[pl.ds(base + off - NBUF * K, K)],
                        sem_w[b]).wait()
                for i in range(K):
                    @plsc.parallel_loop(0, D_MODEL, LANES, unroll=8)
                    def _(j, b=b, i=i):
                        sl = pl.ds(j, LANES)
                        buf_o[b, i, sl] = buf_s[b, i, sl] + buf_t[b, i, sl]
                pltpu.async_copy(
                    buf_o.at[b], out_hbm.at[pl.ds(base + off, K)], sem_w[b])
                @pl.when(o < n_outer - 1)
                def _():
                    off2 = off + NBUF * K
                    pltpu.async_copy(
                        seq_tab_hbm.at[sidx.at[pl.ds(off2, K)]], buf_s.at[b],
                        sem_s[b])
                    pltpu.async_copy(
                        struct_tab_hbm.at[tidx.at[pl.ds(off2, K)]], buf_t.at[b],
                        sem_t[b])
            return 0

        lax.fori_loop(0, n_outer, outer, 0)

        for b in range(NBUF):
            off = (n_chunks - NBUF + b) * K
            pltpu.make_async_copy(
                buf_o.at[b], out_hbm.at[pl.ds(base + off, K)], sem_w[b]).wait()

    return k(seq_tok, struct_tok, seq_table, struct_table)


def kernel(sequence_tokens, structure_tokens, seq_table, struct_table):
    b, s = sequence_tokens.shape
    n = b * s
    seq_tok = sequence_tokens.reshape(n).astype(jnp.int32)
    struct_tok = structure_tokens.reshape(n).astype(jnp.int32)
    out = _gather_add(seq_tok, struct_tok, seq_table, struct_table)
    return out.reshape(b, s, D_MODEL)
